# baseline (speedup 1.0000x reference)
import functools

import numpy as np
import jax
import jax.numpy as jnp
from jax import lax
from jax.experimental import pallas as pl
from jax.experimental.pallas import tpu as pltpu
from jax.experimental.pallas import tpu_sc as plsc

_N = 8388608
_NQ = 5
_L = 16
_NW = 32
_PER_W = _N // _NW
_CHUNK = 8192
_NCHUNK = _PER_W // _CHUNK
_KBITS = 15
_NB = 1 << _KBITS
_SH = 32 - _KBITS
_U = 16
_ZU = 8


def _hist_body(t_hbm, p_hbm, cnt_out, sum_out, cnt_v, sum_v,
               tb0, tb1, pb0, pb1, st0, st1, sp0, sp1):
  wid = lax.axis_index("s") * 2 + lax.axis_index("c")
  base = wid * _PER_W

  zi = jnp.zeros((_L,), jnp.int32)
  zf = jnp.zeros((_L,), jnp.float32)
  ones16 = jnp.ones((_L,), jnp.int32)

  def _zero(i, c):
    for j in range(_ZU):
      off = i * (_L * _ZU) + j * _L
      cnt_v[pl.ds(off, _L)] = zi
      sum_v[pl.ds(off, _L)] = zf
    return c

  lax.fori_loop(0, _NB // (_L * _ZU), _zero, 0)

  tbufs = (tb0, tb1)
  pbufs = (pb0, pb1)
  tsems = (st0, st1)
  psems = (sp0, sp1)
  pend_t = pltpu.async_copy(t_hbm.at[pl.ds(base, _CHUNK)], tb0, st0)
  pend_p = pltpu.async_copy(p_hbm.at[pl.ds(base, _CHUNK)], pb0, sp0)
  for k in range(_NCHUNK):
    tb = tbufs[k % 2]
    pb = pbufs[k % 2]
    nxt_t = nxt_p = None
    if k + 1 < _NCHUNK:
      off = base + (k + 1) * _CHUNK
      nxt_t = pltpu.async_copy(t_hbm.at[pl.ds(off, _CHUNK)],
                               tbufs[(k + 1) % 2], tsems[(k + 1) % 2])
      nxt_p = pltpu.async_copy(p_hbm.at[pl.ds(off, _CHUNK)],
                               pbufs[(k + 1) % 2], psems[(k + 1) % 2])
    pend_t.wait()
    pend_p.wait()

    def _body(i, c):
      ts = []
      ps = []
      for j in range(_U):
        off = i * (_L * _U) + j * _L
        ts.append(tb[pl.ds(off, _L)])
        ps.append(pb[pl.ds(off, _L)])
      idxs = []
      sqs = []
      for t, p in zip(ts, ps):
        d = p - t
        sqs.append(d * d)
        b = lax.bitcast_convert_type(t, jnp.int32)
        m = lax.shift_right_arithmetic(b, 31)
        u = b ^ (m | jnp.int32(-2147483648))
        idxs.append(lax.shift_right_logical(u, _SH))
      for idx in idxs:
        plsc.addupdate_scatter(cnt_v, (idx,), ones16)
      for idx, sq in zip(idxs, sqs):
        plsc.addupdate_scatter(sum_v, (idx,), sq)
      return c

    lax.fori_loop(0, _CHUNK // (_L * _U), _body, 0)
    pend_t = nxt_t
    pend_p = nxt_p

  pltpu.sync_copy(cnt_v, cnt_out.at[wid])
  pltpu.sync_copy(sum_v, sum_out.at[wid])


@functools.cache
def _sc_hist():
  return pl.kernel(
      _hist_body,
      out_type=(jax.ShapeDtypeStruct((_NW, _NB), jnp.int32),
                jax.ShapeDtypeStruct((_NW, _NB), jnp.float32)),
      mesh=plsc.VectorSubcoreMesh(core_axis_name="c", subcore_axis_name="s"),
      compiler_params=pltpu.CompilerParams(needs_layout_passes=False),
      scratch_types=[
          pltpu.VMEM((_NB,), jnp.int32),
          pltpu.VMEM((_NB,), jnp.float32),
          pltpu.VMEM((_CHUNK,), jnp.float32),
          pltpu.VMEM((_CHUNK,), jnp.float32),
          pltpu.VMEM((_CHUNK,), jnp.float32),
          pltpu.VMEM((_CHUNK,), jnp.float32),
          pltpu.SemaphoreType.DMA,
          pltpu.SemaphoreType.DMA,
          pltpu.SemaphoreType.DMA,
          pltpu.SemaphoreType.DMA,
      ],
  )


_MROWS = _NB // 128

_qs_np = np.linspace(0.0, 1.0, _NQ + 1).astype(np.float32)
_pos_np = (_qs_np * np.float32(_N - 1)).astype(np.float32)
_kf_np = np.floor(_pos_np)
_frac_np = _pos_np - _kf_np
_RANKS = tuple(
    int(np.where(_frac_np[i] > 0, _kf_np[i] + 1, _kf_np[i]))
    for i in range(1, _NQ))
_CBINS = (_RANKS[0], _RANKS[1] - _RANKS[0], _RANKS[2] - _RANKS[1],
          _RANKS[3] - _RANKS[2], _N - 1 - _RANKS[3])


def _finish_body(cnt_ref, sum_ref, out_ref):
  f32 = jnp.float32
  hc = jnp.sum(cnt_ref[...].astype(f32), axis=0)
  hs = jnp.sum(sum_ref[...], axis=0)

  e = (lax.broadcasted_iota(jnp.int32, (_MROWS, _MROWS), 1)
       < lax.broadcasted_iota(jnp.int32, (_MROWS, _MROWS), 0)).astype(f32)
  l = (lax.broadcasted_iota(jnp.int32, (128, 128), 0)
       <= lax.broadcasted_iota(jnp.int32, (128, 128), 1)).astype(f32)
  rowpart = lax.dot(e, hc, preferred_element_type=f32)
  rowtot = jnp.sum(rowpart, axis=1, keepdims=True)
  pc = rowtot + lax.dot(hc, l, preferred_element_type=f32)

  flat = (lax.broadcasted_iota(jnp.int32, (_MROWS, 128), 0) * 128
          + lax.broadcasted_iota(jnp.int32, (_MROWS, 128), 1)).astype(f32)

  def split_at(rank_f):
    b = jnp.sum(jnp.where(pc <= rank_f, 1.0, 0.0))
    below = flat < b
    at = flat == b
    nbx = jnp.sum(jnp.where(below, hc, 0.0))
    sbx = jnp.sum(jnp.where(below, hs, 0.0))
    hb = jnp.sum(jnp.where(at, hc, 0.0))
    sb = jnp.sum(jnp.where(at, hs, 0.0))
    return sbx + sb * (rank_f - nbx) / jnp.maximum(hb, 1.0)

  s_below = [split_at(f32(r)) for r in _RANKS]

  btop = jnp.sum(jnp.where(pc <= f32(_N - 1), 1.0, 0.0))
  att = flat == btop
  ntop = jnp.sum(jnp.where(att, hc, 0.0))
  s_topmean = jnp.sum(jnp.where(att, hs, 0.0)) / jnp.maximum(ntop, 1.0)
  total_s = jnp.sum(hs)

  s_bins = [s_below[0], s_below[1] - s_below[0], s_below[2] - s_below[1],
            s_below[3] - s_below[2], total_s - s_below[3] - s_topmean]
  row = lax.broadcasted_iota(jnp.int32, (8, 128), 0)
  lane = lax.broadcasted_iota(jnp.int32, (8, 128), 1)
  acc = jnp.zeros((8, 128), f32)
  for i in range(_NQ):
    acc = acc + jnp.where((row == 0) & (lane == i),
                          s_bins[i] / f32(_CBINS[i]), 0.0)
  out_ref[...] = acc


def _tc_finish(cnt, sm, interpret=False):
  return pl.pallas_call(
      _finish_body,
      out_shape=jax.ShapeDtypeStruct((8, 128), jnp.float32),
      interpret=interpret,
  )(cnt.reshape(_NW, _MROWS, 128), sm.reshape(_NW, _MROWS, 128))


def kernel(predictions, targets, quantile_weights):
  cnt, sm = _sc_hist()(targets, predictions)
  means = _tc_finish(cnt, sm)
  return jnp.sum(quantile_weights * means[0, :_NQ])


# speedup vs baseline: 1.0578x; 1.0578x over previous
import functools

import numpy as np
import jax
import jax.numpy as jnp
from jax import lax
from jax.experimental import pallas as pl
from jax.experimental.pallas import tpu as pltpu
from jax.experimental.pallas import tpu_sc as plsc

_N = 8388608
_NQ = 5
_L = 16
_NW = 32
_PER_W = _N // _NW
_CHUNK = 8192
_NCHUNK = _PER_W // _CHUNK
_KBITS = 15
_NB = 1 << _KBITS
_SH = 32 - _KBITS
_U = 8
_ZU = 8


def _hist_body(t_hbm, p_hbm, cnt_out, sum_out, cnt_v, sum_v,
               tb0, tb1, pb0, pb1, st0, st1, sp0, sp1):
  wid = lax.axis_index("s") * 2 + lax.axis_index("c")
  base = wid * _PER_W

  zi = jnp.zeros((_L,), jnp.int32)
  zf = jnp.zeros((_L,), jnp.float32)
  ones16 = jnp.ones((_L,), jnp.int32)

  def _zero(i, c):
    for j in range(_ZU):
      off = i * (_L * _ZU) + j * _L
      cnt_v[pl.ds(off, _L)] = zi
      sum_v[pl.ds(off, _L)] = zf
    return c

  lax.fori_loop(0, _NB // (_L * _ZU), _zero, 0)

  tbufs = (tb0, tb1)
  pbufs = (pb0, pb1)
  tsems = (st0, st1)
  psems = (sp0, sp1)
  pend_t = pltpu.async_copy(t_hbm.at[pl.ds(base, _CHUNK)], tb0, st0)
  pend_p = pltpu.async_copy(p_hbm.at[pl.ds(base, _CHUNK)], pb0, sp0)
  for k in range(_NCHUNK):
    tb = tbufs[k % 2]
    pb = pbufs[k % 2]
    nxt_t = nxt_p = None
    if k + 1 < _NCHUNK:
      off = base + (k + 1) * _CHUNK
      nxt_t = pltpu.async_copy(t_hbm.at[pl.ds(off, _CHUNK)],
                               tbufs[(k + 1) % 2], tsems[(k + 1) % 2])
      nxt_p = pltpu.async_copy(p_hbm.at[pl.ds(off, _CHUNK)],
                               pbufs[(k + 1) % 2], psems[(k + 1) % 2])
    pend_t.wait()
    pend_p.wait()

    def _body(i, c):
      ts = []
      ps = []
      for j in range(_U):
        off = i * (_L * _U) + j * _L
        ts.append(tb[pl.ds(off, _L)])
        ps.append(pb[pl.ds(off, _L)])
      idxs = []
      sqs = []
      for t, p in zip(ts, ps):
        d = p - t
        sqs.append(d * d)
        b = lax.bitcast_convert_type(t, jnp.int32)
        m = lax.shift_right_arithmetic(b, 31)
        u = b ^ (m | jnp.int32(-2147483648))
        idxs.append(lax.shift_right_logical(u, _SH))
      for idx in idxs:
        plsc.addupdate_scatter(cnt_v, (idx,), ones16)
      for idx, sq in zip(idxs, sqs):
        plsc.addupdate_scatter(sum_v, (idx,), sq)
      return c

    lax.fori_loop(0, _CHUNK // (_L * _U), _body, 0)
    pend_t = nxt_t
    pend_p = nxt_p

  pltpu.sync_copy(cnt_v, cnt_out.at[wid])
  pltpu.sync_copy(sum_v, sum_out.at[wid])


@functools.cache
def _sc_hist():
  return pl.kernel(
      _hist_body,
      out_type=(jax.ShapeDtypeStruct((_NW, _NB), jnp.int32),
                jax.ShapeDtypeStruct((_NW, _NB), jnp.float32)),
      mesh=plsc.VectorSubcoreMesh(core_axis_name="c", subcore_axis_name="s"),
      compiler_params=pltpu.CompilerParams(needs_layout_passes=False),
      scratch_types=[
          pltpu.VMEM((_NB,), jnp.int32),
          pltpu.VMEM((_NB,), jnp.float32),
          pltpu.VMEM((_CHUNK,), jnp.float32),
          pltpu.VMEM((_CHUNK,), jnp.float32),
          pltpu.VMEM((_CHUNK,), jnp.float32),
          pltpu.VMEM((_CHUNK,), jnp.float32),
          pltpu.SemaphoreType.DMA,
          pltpu.SemaphoreType.DMA,
          pltpu.SemaphoreType.DMA,
          pltpu.SemaphoreType.DMA,
      ],
  )


_MROWS = _NB // 128

_qs_np = np.linspace(0.0, 1.0, _NQ + 1).astype(np.float32)
_pos_np = (_qs_np * np.float32(_N - 1)).astype(np.float32)
_kf_np = np.floor(_pos_np)
_frac_np = _pos_np - _kf_np
_RANKS = tuple(
    int(np.where(_frac_np[i] > 0, _kf_np[i] + 1, _kf_np[i]))
    for i in range(1, _NQ))
_CBINS = (_RANKS[0], _RANKS[1] - _RANKS[0], _RANKS[2] - _RANKS[1],
          _RANKS[3] - _RANKS[2], _N - 1 - _RANKS[3])


def _finish_body(qw_ref, cnt_ref, sum_ref, out_ref):
  f32 = jnp.float32
  hc = jnp.sum(cnt_ref[...].astype(f32), axis=0)
  hs = jnp.sum(sum_ref[...], axis=0)

  e = (lax.broadcasted_iota(jnp.int32, (_MROWS, _MROWS), 1)
       < lax.broadcasted_iota(jnp.int32, (_MROWS, _MROWS), 0)).astype(f32)
  l = (lax.broadcasted_iota(jnp.int32, (128, 128), 0)
       <= lax.broadcasted_iota(jnp.int32, (128, 128), 1)).astype(f32)
  rowpart = lax.dot(e, hc, preferred_element_type=f32)
  rowtot = jnp.sum(rowpart, axis=1, keepdims=True)
  pc = rowtot + lax.dot(hc, l, preferred_element_type=f32)

  flat = (lax.broadcasted_iota(jnp.int32, (_MROWS, 128), 0) * 128
          + lax.broadcasted_iota(jnp.int32, (_MROWS, 128), 1)).astype(f32)

  def split_at(rank_f):
    b = jnp.sum(jnp.where(pc <= rank_f, 1.0, 0.0))
    below = flat < b
    at = flat == b
    nbx = jnp.sum(jnp.where(below, hc, 0.0))
    sbx = jnp.sum(jnp.where(below, hs, 0.0))
    hb = jnp.sum(jnp.where(at, hc, 0.0))
    sb = jnp.sum(jnp.where(at, hs, 0.0))
    return sbx + sb * (rank_f - nbx) / jnp.maximum(hb, 1.0)

  s_below = [split_at(f32(r)) for r in _RANKS]

  btop = jnp.sum(jnp.where(pc <= f32(_N - 1), 1.0, 0.0))
  att = flat == btop
  ntop = jnp.sum(jnp.where(att, hc, 0.0))
  s_topmean = jnp.sum(jnp.where(att, hs, 0.0)) / jnp.maximum(ntop, 1.0)
  total_s = jnp.sum(hs)

  s_bins = [s_below[0], s_below[1] - s_below[0], s_below[2] - s_below[1],
            s_below[3] - s_below[2], total_s - s_below[3] - s_topmean]
  loss = f32(0.0)
  for i in range(_NQ):
    loss = loss + qw_ref[i] * (s_bins[i] / f32(_CBINS[i]))
  out_ref[...] = jnp.full((1, 1), 1.0, f32) * loss


def _tc_finish(qw, cnt, sm, interpret=False):
  return pl.pallas_call(
      _finish_body,
      in_specs=[
          pl.BlockSpec(memory_space=pltpu.SMEM),
          pl.BlockSpec(memory_space=pltpu.VMEM),
          pl.BlockSpec(memory_space=pltpu.VMEM),
      ],
      out_shape=jax.ShapeDtypeStruct((1, 1), jnp.float32),
      interpret=interpret,
  )(qw, cnt.reshape(_NW, _MROWS, 128), sm.reshape(_NW, _MROWS, 128))


def kernel(predictions, targets, quantile_weights):
  cnt, sm = _sc_hist()(targets, predictions)
  out = _tc_finish(quantile_weights, cnt, sm)
  return out[0, 0]


# speedup vs baseline: 1.1566x; 1.0933x over previous
import functools

import numpy as np
import jax
import jax.numpy as jnp
from jax import lax
from jax.experimental import pallas as pl
from jax.experimental.pallas import tpu as pltpu
from jax.experimental.pallas import tpu_sc as plsc

_N = 8388608
_NQ = 5
_L = 16
_NW = 32
_PER_W = _N // _NW
_CHUNK = 16384
_NCHUNK = _PER_W // _CHUNK
_KBITS = 14
_NB = 1 << _KBITS
_SH = 32 - _KBITS
_U = 8
_ZU = 8


def _hist_body(t_hbm, p_hbm, cnt_out, sum_out, cnt_v, sum_v,
               tb0, tb1, pb0, pb1, st0, st1, sp0, sp1):
  wid = lax.axis_index("s") * 2 + lax.axis_index("c")
  base = wid * _PER_W

  zi = jnp.zeros((_L,), jnp.int32)
  zf = jnp.zeros((_L,), jnp.float32)
  ones16 = jnp.ones((_L,), jnp.int32)

  def _zero(i, c):
    for j in range(_ZU):
      off = i * (_L * _ZU) + j * _L
      cnt_v[pl.ds(off, _L)] = zi
      sum_v[pl.ds(off, _L)] = zf
    return c

  lax.fori_loop(0, _NB // (_L * _ZU), _zero, 0)

  tbufs = (tb0, tb1)
  pbufs = (pb0, pb1)
  tsems = (st0, st1)
  psems = (sp0, sp1)
  pend_t = pltpu.async_copy(t_hbm.at[pl.ds(base, _CHUNK)], tb0, st0)
  pend_p = pltpu.async_copy(p_hbm.at[pl.ds(base, _CHUNK)], pb0, sp0)
  for k in range(_NCHUNK):
    tb = tbufs[k % 2]
    pb = pbufs[k % 2]
    nxt_t = nxt_p = None
    if k + 1 < _NCHUNK:
      off = base + (k + 1) * _CHUNK
      nxt_t = pltpu.async_copy(t_hbm.at[pl.ds(off, _CHUNK)],
                               tbufs[(k + 1) % 2], tsems[(k + 1) % 2])
      nxt_p = pltpu.async_copy(p_hbm.at[pl.ds(off, _CHUNK)],
                               pbufs[(k + 1) % 2], psems[(k + 1) % 2])
    pend_t.wait()
    pend_p.wait()

    def _body(i, c):
      ts = []
      ps = []
      for j in range(_U):
        off = i * (_L * _U) + j * _L
        ts.append(tb[pl.ds(off, _L)])
        ps.append(pb[pl.ds(off, _L)])
      idxs = []
      sqs = []
      for t, p in zip(ts, ps):
        d = p - t
        sqs.append(d * d)
        b = lax.bitcast_convert_type(t, jnp.int32)
        m = lax.shift_right_arithmetic(b, 31)
        u = b ^ (m | jnp.int32(-2147483648))
        idxs.append(lax.shift_right_logical(u, _SH))
      for idx in idxs:
        plsc.addupdate_scatter(cnt_v, (idx,), ones16)
      for idx, sq in zip(idxs, sqs):
        plsc.addupdate_scatter(sum_v, (idx,), sq)
      return c

    lax.fori_loop(0, _CHUNK // (_L * _U), _body, 0)
    pend_t = nxt_t
    pend_p = nxt_p

  pltpu.sync_copy(cnt_v, cnt_out.at[wid])
  pltpu.sync_copy(sum_v, sum_out.at[wid])


@functools.cache
def _sc_hist():
  return pl.kernel(
      _hist_body,
      out_type=(jax.ShapeDtypeStruct((_NW, _NB), jnp.int32),
                jax.ShapeDtypeStruct((_NW, _NB), jnp.float32)),
      mesh=plsc.VectorSubcoreMesh(core_axis_name="c", subcore_axis_name="s"),
      compiler_params=pltpu.CompilerParams(needs_layout_passes=False),
      scratch_types=[
          pltpu.VMEM((_NB,), jnp.int32),
          pltpu.VMEM((_NB,), jnp.float32),
          pltpu.VMEM((_CHUNK,), jnp.float32),
          pltpu.VMEM((_CHUNK,), jnp.float32),
          pltpu.VMEM((_CHUNK,), jnp.float32),
          pltpu.VMEM((_CHUNK,), jnp.float32),
          pltpu.SemaphoreType.DMA,
          pltpu.SemaphoreType.DMA,
          pltpu.SemaphoreType.DMA,
          pltpu.SemaphoreType.DMA,
      ],
  )


_MROWS = _NB // 128

_qs_np = np.linspace(0.0, 1.0, _NQ + 1).astype(np.float32)
_pos_np = (_qs_np * np.float32(_N - 1)).astype(np.float32)
_kf_np = np.floor(_pos_np)
_frac_np = _pos_np - _kf_np
_RANKS = tuple(
    int(np.where(_frac_np[i] > 0, _kf_np[i] + 1, _kf_np[i]))
    for i in range(1, _NQ))
_CBINS = (_RANKS[0], _RANKS[1] - _RANKS[0], _RANKS[2] - _RANKS[1],
          _RANKS[3] - _RANKS[2], _N - 1 - _RANKS[3])


def _finish_body(qw_ref, cnt_ref, sum_ref, out_ref):
  f32 = jnp.float32
  hc = jnp.sum(cnt_ref[...].astype(f32), axis=0)
  hs = jnp.sum(sum_ref[...], axis=0)

  e = (lax.broadcasted_iota(jnp.int32, (_MROWS, _MROWS), 1)
       < lax.broadcasted_iota(jnp.int32, (_MROWS, _MROWS), 0)).astype(f32)
  l = (lax.broadcasted_iota(jnp.int32, (128, 128), 0)
       <= lax.broadcasted_iota(jnp.int32, (128, 128), 1)).astype(f32)
  rowpart = lax.dot(e, hc, preferred_element_type=f32)
  rowtot = jnp.sum(rowpart, axis=1, keepdims=True)
  pc = rowtot + lax.dot(hc, l, preferred_element_type=f32)

  flat = (lax.broadcasted_iota(jnp.int32, (_MROWS, 128), 0) * 128
          + lax.broadcasted_iota(jnp.int32, (_MROWS, 128), 1)).astype(f32)

  def split_at(rank_f):
    b = jnp.sum(jnp.where(pc <= rank_f, 1.0, 0.0))
    below = flat < b
    at = flat == b
    nbx = jnp.sum(jnp.where(below, hc, 0.0))
    sbx = jnp.sum(jnp.where(below, hs, 0.0))
    hb = jnp.sum(jnp.where(at, hc, 0.0))
    sb = jnp.sum(jnp.where(at, hs, 0.0))
    return sbx + sb * (rank_f - nbx) / jnp.maximum(hb, 1.0)

  s_below = [split_at(f32(r)) for r in _RANKS]

  btop = jnp.sum(jnp.where(pc <= f32(_N - 1), 1.0, 0.0))
  att = flat == btop
  ntop = jnp.sum(jnp.where(att, hc, 0.0))
  s_topmean = jnp.sum(jnp.where(att, hs, 0.0)) / jnp.maximum(ntop, 1.0)
  total_s = jnp.sum(hs)

  s_bins = [s_below[0], s_below[1] - s_below[0], s_below[2] - s_below[1],
            s_below[3] - s_below[2], total_s - s_below[3] - s_topmean]
  loss = f32(0.0)
  for i in range(_NQ):
    loss = loss + qw_ref[i] * (s_bins[i] / f32(_CBINS[i]))
  out_ref[...] = jnp.full((1, 1), 1.0, f32) * loss


def _tc_finish(qw, cnt, sm, interpret=False):
  return pl.pallas_call(
      _finish_body,
      in_specs=[
          pl.BlockSpec(memory_space=pltpu.SMEM),
          pl.BlockSpec(memory_space=pltpu.VMEM),
          pl.BlockSpec(memory_space=pltpu.VMEM),
      ],
      out_shape=jax.ShapeDtypeStruct((1, 1), jnp.float32),
      interpret=interpret,
  )(qw, cnt.reshape(_NW, _MROWS, 128), sm.reshape(_NW, _MROWS, 128))


def kernel(predictions, targets, quantile_weights):
  cnt, sm = _sc_hist()(targets, predictions)
  out = _tc_finish(quantile_weights, cnt, sm)
  return out[0, 0]
